# baseline (speedup 1.0000x reference)
import functools

import jax
import jax.numpy as jnp
from jax import lax
from jax.experimental import pallas as pl
from jax.experimental.pallas import tpu as pltpu

_EPS = 1e-5


def _upconv_kernel(x_ref, w_ref, b_ref, o_ref):
    R, _, W, K2 = o_ref.shape
    y = jnp.dot(x_ref[...], w_ref[...],
                preferred_element_type=jnp.float32) + b_ref[...]
    o_ref[:, 0, :, :] = y[:, :K2].reshape(R, W, K2).astype(o_ref.dtype)
    o_ref[:, 1, :, :] = y[:, K2:].reshape(R, W, K2).astype(o_ref.dtype)


def _double_conv_kernel(xa_ref, xb_ref,
                        w1a_ref, w1b_ref, s1_ref, b1_ref,
                        w2_ref, s2_ref, b2_ref, o_ref, *, oh, ow):
    M = xa_ref.shape[1]

    flat = lax.broadcasted_iota(jnp.int32, (M, 1), 0)
    col = flat & (ow - 1) if (ow & (ow - 1)) == 0 else flat % ow
    row_m = [flat >= ow, None, flat < (oh - 1) * ow]
    col_m = [col >= 1, None, col < (ow - 1)]
    tap_mask = []
    for dy in range(3):
        for dx in range(3):
            ms = [m for m in (row_m[dy], col_m[dx]) if m is not None]
            if not ms:
                tap_mask.append(None)
            elif len(ms) == 1:
                tap_mask.append(ms[0])
            else:
                tap_mask.append(jnp.logical_and(ms[0], ms[1]))

    def conv3x3(parts):
        acc = jnp.zeros((M, parts[0][1].shape[2]), jnp.float32)
        for dy in range(3):
            for dx in range(3):
                tap = dy * 3 + dx
                off = (dy - 1) * ow + (dx - 1)
                p = jnp.dot(parts[0][0], parts[0][1][tap],
                            preferred_element_type=jnp.float32)
                for x_, w_ in parts[1:]:
                    p += jnp.dot(x_, w_[tap],
                                 preferred_element_type=jnp.float32)
                shift = (-off) % M
                if shift:
                    p = pltpu.roll(p, shift, axis=0)
                if tap_mask[tap] is not None:
                    p = jnp.where(tap_mask[tap], p, 0.0)
                acc = acc + p
        return acc

    h1 = jnp.maximum(conv3x3([(xa_ref[0], w1a_ref), (xb_ref[0], w1b_ref)])
                     * s1_ref[...] + b1_ref[...], 0.0)
    h1 = h1.astype(jnp.bfloat16)
    y2 = jnp.maximum(conv3x3([(h1, w2_ref)])
                     * s2_ref[...] + b2_ref[...], 0.0)
    o_ref[0] = y2


def _fold_bn(conv_b, gamma, beta, rmean, rvar):
    scale = gamma / jnp.sqrt(rvar + _EPS)
    bias = beta + scale * (conv_b - rmean)
    return scale, bias


def kernel(x1, x2, up_w, up_b, c1_w, c1_b, bn1_g, bn1_b, bn1_m, bn1_v,
           c2_w, c2_b, bn2_g, bn2_b, bn2_m, bn2_v):
    N, C, H1, W1 = x1.shape
    oh, ow = 2 * H1, 2 * W1
    M = oh * ow
    Cmid = c1_w.shape[0]
    Cout = c2_w.shape[0]

    x1f = jnp.transpose(x1, (0, 2, 3, 1)).reshape(N * H1 * W1, C)
    x1f = x1f.astype(jnp.bfloat16)
    x2f = jnp.transpose(x2, (0, 2, 3, 1)).reshape(N, M, C)
    x2f = x2f.astype(jnp.bfloat16)

    wup = jnp.transpose(up_w, (0, 2, 3, 1)).reshape(C, 4 * C)
    wup = wup.astype(jnp.bfloat16)
    bup = jnp.tile(up_b, 4)[None, :]

    rows_total = N * H1
    R = 8
    up_out = pl.pallas_call(
        _upconv_kernel,
        out_shape=jax.ShapeDtypeStruct((rows_total, 2, W1, 2 * C),
                                       jnp.bfloat16),
        grid=(rows_total // R,),
        in_specs=[
            pl.BlockSpec((R * W1, C), lambda i: (i, 0)),
            pl.BlockSpec((C, 4 * C), lambda i: (0, 0)),
            pl.BlockSpec((1, 4 * C), lambda i: (0, 0)),
        ],
        out_specs=pl.BlockSpec((R, 2, W1, 2 * C), lambda i: (i, 0, 0, 0)),
        compiler_params=pltpu.CompilerParams(
            dimension_semantics=("parallel",),
            vmem_limit_bytes=64 * 1024 * 1024,
        ),
    )(x1f, wup, bup)
    x1u = up_out.reshape(N, M, C)

    w1 = jnp.transpose(c1_w, (2, 3, 1, 0)).reshape(9, 2 * C, Cmid)
    w1a = w1[:, :C].astype(jnp.bfloat16)
    w1b = w1[:, C:].astype(jnp.bfloat16)
    w2 = jnp.transpose(c2_w, (2, 3, 1, 0)).reshape(9, Cmid, Cout)
    w2 = w2.astype(jnp.bfloat16)
    s1, b1 = _fold_bn(c1_b, bn1_g, bn1_b, bn1_m, bn1_v)
    s2, b2 = _fold_bn(c2_b, bn2_g, bn2_b, bn2_m, bn2_v)

    body = functools.partial(_double_conv_kernel, oh=oh, ow=ow)
    out = pl.pallas_call(
        body,
        out_shape=jax.ShapeDtypeStruct((N, M, Cout), jnp.float32),
        grid=(N,),
        in_specs=[
            pl.BlockSpec((1, M, C), lambda n: (n, 0, 0)),
            pl.BlockSpec((1, M, C), lambda n: (n, 0, 0)),
            pl.BlockSpec((9, C, Cmid), lambda n: (0, 0, 0)),
            pl.BlockSpec((9, C, Cmid), lambda n: (0, 0, 0)),
            pl.BlockSpec((1, Cmid), lambda n: (0, 0)),
            pl.BlockSpec((1, Cmid), lambda n: (0, 0)),
            pl.BlockSpec((9, Cmid, Cout), lambda n: (0, 0, 0)),
            pl.BlockSpec((1, Cout), lambda n: (0, 0)),
            pl.BlockSpec((1, Cout), lambda n: (0, 0)),
        ],
        out_specs=pl.BlockSpec((1, M, Cout), lambda n: (n, 0, 0)),
        compiler_params=pltpu.CompilerParams(
            dimension_semantics=("parallel",),
            vmem_limit_bytes=64 * 1024 * 1024,
        ),
    )(x2f, x1u, w1a, w1b, s1[None, :], b1[None, :],
      w2, s2[None, :], b2[None, :])

    out = out.reshape(N, oh, ow, Cout)
    return jnp.transpose(out, (0, 3, 1, 2))


# speedup vs baseline: 1.2250x; 1.2250x over previous
import functools

import jax
import jax.numpy as jnp
from jax import lax
from jax.experimental import pallas as pl
from jax.experimental.pallas import tpu as pltpu

_EPS = 1e-5


def _upconv_kernel(x_ref, w_ref, b_ref, o_ref):
    R, _, W, K2 = o_ref.shape
    y = jnp.dot(x_ref[...], w_ref[...],
                preferred_element_type=jnp.float32) + b_ref[...]
    o_ref[:, 0, :, :] = y[:, :K2].reshape(R, W, K2).astype(o_ref.dtype)
    o_ref[:, 1, :, :] = y[:, K2:].reshape(R, W, K2).astype(o_ref.dtype)


def _double_conv_kernel(xa_ref, xb_ref, w1s_ref, b1_ref,
                        w2s_ref, b2_ref, o_ref,
                        sbuf1, sbuf2, *, oh, ow):
    M = xa_ref.shape[1]
    C = xa_ref.shape[2]
    Cmid = b1_ref.shape[1]

    flat = lax.broadcasted_iota(jnp.int32, (M, 1), 0)
    col = flat & (ow - 1) if (ow & (ow - 1)) == 0 else flat % ow
    m_l = col >= 1
    m_r = col < (ow - 1)

    def shifted(x):
        z = jnp.zeros_like(x)
        return (jnp.where(m_l, pltpu.roll(x, 1, axis=0), z),
                jnp.where(m_r, pltpu.roll(x, M - 1, axis=0), z))

    def conv(sbuf, pieces, ws_ref, nlanes):
        sbuf[0:ow, :] = jnp.zeros((ow, nlanes), jnp.bfloat16)
        sbuf[ow + M:, :] = jnp.zeros((ow, nlanes), jnp.bfloat16)
        for k, piece in enumerate(pieces):
            sbuf[pl.ds(ow, M), k * C:(k + 1) * C] = piece
        acc = jnp.dot(sbuf[pl.ds(0, M), :], ws_ref[0],
                      preferred_element_type=jnp.float32)
        acc += jnp.dot(sbuf[pl.ds(ow, M), :], ws_ref[1],
                       preferred_element_type=jnp.float32)
        acc += jnp.dot(sbuf[pl.ds(2 * ow, M), :], ws_ref[2],
                       preferred_element_type=jnp.float32)
        return acc

    xa = xa_ref[0]
    xb = xb_ref[0]
    xa_m, xa_p = shifted(xa)
    xb_m, xb_p = shifted(xb)
    acc1 = conv(sbuf1, (xa_m, xa, xa_p, xb_m, xb, xb_p), w1s_ref, 6 * C)
    h1 = jnp.maximum(acc1 + b1_ref[...], 0.0).astype(jnp.bfloat16)

    h1_m, h1_p = shifted(h1)
    acc2 = conv(sbuf2, (h1_m, h1, h1_p), w2s_ref, 3 * Cmid)
    y2 = jnp.maximum(acc2 + b2_ref[...], 0.0)
    o_ref[0] = y2.astype(o_ref.dtype)


def _fold_bn(conv_b, gamma, beta, rmean, rvar):
    scale = gamma / jnp.sqrt(rvar + _EPS)
    bias = beta + scale * (conv_b - rmean)
    return scale, bias


def kernel(x1, x2, up_w, up_b, c1_w, c1_b, bn1_g, bn1_b, bn1_m, bn1_v,
           c2_w, c2_b, bn2_g, bn2_b, bn2_m, bn2_v):
    N, C, H1, W1 = x1.shape
    oh, ow = 2 * H1, 2 * W1
    M = oh * ow
    Cmid = c1_w.shape[0]
    Cout = c2_w.shape[0]

    x1f = jnp.transpose(x1, (0, 2, 3, 1)).reshape(N * H1 * W1, C)
    x1f = x1f.astype(jnp.bfloat16)
    x2f = jnp.transpose(x2, (0, 2, 3, 1)).reshape(N, M, C)
    x2f = x2f.astype(jnp.bfloat16)

    wup = jnp.transpose(up_w, (0, 2, 3, 1)).reshape(C, 4 * C)
    wup = wup.astype(jnp.bfloat16)
    bup = jnp.tile(up_b, 4)[None, :]

    rows_total = N * H1
    R = 64
    up_out = pl.pallas_call(
        _upconv_kernel,
        out_shape=jax.ShapeDtypeStruct((rows_total, 2, W1, 2 * C),
                                       jnp.bfloat16),
        grid=(rows_total // R,),
        in_specs=[
            pl.BlockSpec((R * W1, C), lambda i: (i, 0)),
            pl.BlockSpec((C, 4 * C), lambda i: (0, 0)),
            pl.BlockSpec((1, 4 * C), lambda i: (0, 0)),
        ],
        out_specs=pl.BlockSpec((R, 2, W1, 2 * C), lambda i: (i, 0, 0, 0)),
        compiler_params=pltpu.CompilerParams(
            dimension_semantics=("parallel",),
            vmem_limit_bytes=64 * 1024 * 1024,
        ),
    )(x1f, wup, bup)
    x1u = up_out.reshape(N, M, C)

    s1, b1 = _fold_bn(c1_b, bn1_g, bn1_b, bn1_m, bn1_v)
    s2, b2 = _fold_bn(c2_b, bn2_g, bn2_b, bn2_m, bn2_v)

    w1 = jnp.transpose(c1_w, (2, 3, 1, 0)).reshape(3, 3, 2 * C, Cmid)
    w1s = jnp.concatenate(
        [w1[:, 0, :C], w1[:, 1, :C], w1[:, 2, :C],
         w1[:, 0, C:], w1[:, 1, C:], w1[:, 2, C:]], axis=1)
    w1s = (w1s * s1[None, None, :]).astype(jnp.bfloat16)
    w2 = jnp.transpose(c2_w, (2, 3, 1, 0)).reshape(3, 3, Cmid, Cout)
    w2s = jnp.concatenate([w2[:, 0], w2[:, 1], w2[:, 2]], axis=1)
    w2s = (w2s * s2[None, None, :]).astype(jnp.bfloat16)

    body = functools.partial(_double_conv_kernel, oh=oh, ow=ow)
    out = pl.pallas_call(
        body,
        out_shape=jax.ShapeDtypeStruct((N, M, Cout), jnp.bfloat16),
        grid=(N,),
        in_specs=[
            pl.BlockSpec((1, M, C), lambda n: (n, 0, 0)),
            pl.BlockSpec((1, M, C), lambda n: (n, 0, 0)),
            pl.BlockSpec((3, 6 * C, Cmid), lambda n: (0, 0, 0)),
            pl.BlockSpec((1, Cmid), lambda n: (0, 0)),
            pl.BlockSpec((3, 3 * Cmid, Cout), lambda n: (0, 0, 0)),
            pl.BlockSpec((1, Cout), lambda n: (0, 0)),
        ],
        out_specs=pl.BlockSpec((1, M, Cout), lambda n: (n, 0, 0)),
        scratch_shapes=[
            pltpu.VMEM((M + 2 * ow, 6 * C), jnp.bfloat16),
            pltpu.VMEM((M + 2 * ow, 3 * Cmid), jnp.bfloat16),
        ],
        compiler_params=pltpu.CompilerParams(
            dimension_semantics=("parallel",),
            vmem_limit_bytes=64 * 1024 * 1024,
        ),
    )(x2f, x1u, w1s, b1[None, :], w2s, b2[None, :])

    out = out.reshape(N, oh, ow, Cout)
    return jnp.transpose(out, (0, 3, 1, 2)).astype(jnp.float32)
